# baseline (speedup 1.0000x reference)
import functools
import math

import jax
import jax.numpy as jnp
from jax import lax
from jax.experimental import pallas as pl
from jax.experimental.pallas import tpu as pltpu
from jax.experimental.pallas import tpu_sc as plsc

B = 8
N = 12500
K = 32
C = 3
NUM_CLASSES = 40
TOTAL = B * N
E = 3200000
EPS = 1e-12

NC = 2
NS = 16
NW = NC * NS
NPAD = 96
PT = TOTAL + NPAD
SL = PT // NS

CH = 2048
CROWS = CH // 128
NCHUNK = 1568
E_PAD = NCHUNK * CH
CPW = NCHUNK // NW

H1 = 1024
H2 = 512
BN_SCALE = float(1.0 / math.sqrt(1.0 + 1e-5))


QBITS_XY = 11
QBITS_Z = 10
QLIM = 6.0
QSTEP_XY = 2.0 * QLIM / float((1 << QBITS_XY) - 1)
QSTEP_Z = 2.0 * QLIM / float((1 << QBITS_Z) - 1)


def _sc_edge_kernel(pq_h, row_h, col_h, out_sum, out_cnt,
                    pq_sp, sum_sp, cnt_sp,
                    idxr_a, idxc_a, wr_a, wc_a, dist_a,
                    idxr_b, idxc_b, wr_b, wc_b, dist_b,
                    ones_v, stage_v, stage_iv, sem):
    cid = lax.axis_index("c")
    sid = lax.axis_index("s")
    wid = sid * NC + cid

    def _fill_one(i, _):
        ones_v[pl.ds(i * 16, 16)] = jnp.ones((16,), jnp.float32)
        return 0
    lax.fori_loop(0, CH // 16, _fill_one, 0)

    off = sid * SL
    sl = pl.ds(off, SL)
    pltpu.sync_copy(pq_h.at[sl], stage_iv)
    pltpu.sync_copy(stage_iv, pq_sp.at[sl])

    def _fill_z(i, _):
        stage_v[pl.ds(i * 16, 16)] = jnp.zeros((16,), jnp.float32)
        return 0
    lax.fori_loop(0, SL // 16, _fill_z, 0)
    pltpu.sync_copy(stage_v, sum_sp.at[sl])
    pltpu.sync_copy(stage_v, cnt_sp.at[sl])
    plsc.subcore_barrier()

    mxy = jnp.full((16,), (1 << QBITS_XY) - 1, jnp.int32)
    mz = jnp.full((16,), (1 << QBITS_Z) - 1, jnp.int32)
    expo = jnp.full((16,), 0x4B000000, jnp.int32)

    def _i2f(q):
        return lax.bitcast_convert_type(expo | q, jnp.float32)

    def _do_chunk(g, idxr_v, idxc_v, wr_v, wc_v, dist_v):
        eoff = (wid * CPW + g) * CH
        pltpu.sync_copy(row_h.at[pl.ds(eoff, CH)], idxr_v)
        pltpu.sync_copy(col_h.at[pl.ds(eoff, CH)], idxc_v)
        descs = [
            pltpu.async_copy(pq_sp.at[idxr_v], wr_v, sem),
            pltpu.async_copy(pq_sp.at[idxc_v], wc_v, sem),
        ]
        for d in descs:
            d.wait()

        def _dist(i, _):
            s = pl.ds(i * 16, 16)
            wr = wr_v[s]
            wc = wc_v[s]
            dx = (_i2f(lax.shift_right_logical(wr, 21) & mxy)
                  - _i2f(lax.shift_right_logical(wc, 21) & mxy)) * QSTEP_XY
            dy = (_i2f(lax.shift_right_logical(wr, 10) & mxy)
                  - _i2f(lax.shift_right_logical(wc, 10) & mxy)) * QSTEP_XY
            dz = (_i2f(wr & mz) - _i2f(wc & mz)) * QSTEP_Z
            d2 = dx * dx + dy * dy + dz * dz
            ib = lax.bitcast_convert_type(d2, jnp.int32)
            y = lax.bitcast_convert_type(
                jnp.full((16,), 0x5F3759DF, jnp.int32) - (ib >> 1), jnp.float32)
            h = d2 * 0.5
            y = y * (1.5 - h * y * y)
            y = y * (1.5 - h * y * y)
            y = y * (1.5 - h * y * y)
            dist_v[s] = d2 * y
            return 0
        lax.fori_loop(0, CH // 16, _dist, 0)

        pltpu.sync_copy(dist_v, sum_sp.at[idxr_v], add=True)
        pltpu.sync_copy(ones_v, cnt_sp.at[idxr_v], add=True)

    def _chunk_pair(i, _):
        _do_chunk(2 * i, idxr_a, idxc_a, wr_a, wc_a, dist_a)
        _do_chunk(2 * i + 1, idxr_b, idxc_b, wr_b, wc_b, dist_b)
        return 0

    lax.fori_loop(0, CPW // 2, _chunk_pair, 0)
    if CPW % 2:
        _do_chunk(CPW - 1, idxr_a, idxc_a, wr_a, wc_a, dist_a)
    plsc.subcore_barrier()

    oo = cid * PT + off
    pltpu.sync_copy(sum_sp.at[sl], stage_v)
    pltpu.sync_copy(stage_v, out_sum.at[pl.ds(oo, SL)])
    pltpu.sync_copy(cnt_sp.at[sl], stage_v)
    pltpu.sync_copy(stage_v, out_cnt.at[pl.ds(oo, SL)])


_sc_edge = pl.kernel(
    _sc_edge_kernel,
    out_type=(jax.ShapeDtypeStruct((NC * PT,), jnp.float32),
              jax.ShapeDtypeStruct((NC * PT,), jnp.float32)),
    mesh=plsc.VectorSubcoreMesh(core_axis_name="c", subcore_axis_name="s",
                                num_cores=NC, num_subcores=NS),
    scratch_types=(
        pltpu.VMEM_SHARED((PT,), jnp.int32),
        pltpu.VMEM_SHARED((PT,), jnp.float32),
        pltpu.VMEM_SHARED((PT,), jnp.float32),
        pltpu.VMEM((CH,), jnp.int32),
        pltpu.VMEM((CH,), jnp.int32),
        pltpu.VMEM((CH,), jnp.int32),
        pltpu.VMEM((CH,), jnp.int32),
        pltpu.VMEM((CH,), jnp.float32),
        pltpu.VMEM((CH,), jnp.int32),
        pltpu.VMEM((CH,), jnp.int32),
        pltpu.VMEM((CH,), jnp.int32),
        pltpu.VMEM((CH,), jnp.int32),
        pltpu.VMEM((CH,), jnp.float32),
        pltpu.VMEM((CH,), jnp.float32),
        pltpu.VMEM((SL,), jnp.float32),
        pltpu.VMEM((SL,), jnp.int32),
        pltpu.SemaphoreType.DMA,
    ),
)


def _tc_dense_kernel(sum_ref, cnt_ref, pos_ref, phi_ref, sf_ref,
                     w1_ref, b1_ref, w2_ref, b2_ref, w3_ref, b3_ref,
                     logp_ref, w_ref):
    s2 = sum_ref[...][0]
    c2 = cnt_ref[...][0]
    s = s2[0:1] + s2[1:2]
    c = c2[0:1] + c2[1:2]
    mean = jnp.where(c > 0, s / jnp.maximum(c, 1.0), 0.0)
    tot = jnp.sum(mean)
    wq = mean * (float(N) / (tot + EPS))
    w_ref[...] = wq[None]

    U = pos_ref[...][0] * wq
    Pb = phi_ref[...][0]
    F = lax.dot_general(U, Pb, (((1,), (1,)), ((), ())),
                        preferred_element_type=jnp.float32)
    Y = jnp.abs(F * sf_ref[...])

    W1v = w1_ref[...]
    h = lax.dot_general(Y[0:1], W1v[0], (((1,), (0,)), ((), ())),
                        preferred_element_type=jnp.float32)
    h = h + lax.dot_general(Y[1:2], W1v[1], (((1,), (0,)), ((), ())),
                            preferred_element_type=jnp.float32)
    h = h + lax.dot_general(Y[2:3], W1v[2], (((1,), (0,)), ((), ())),
                            preferred_element_type=jnp.float32)
    h = jnp.maximum((h + b1_ref[...]) * BN_SCALE, 0.0)

    h2 = lax.dot_general(h, w2_ref[...], (((1,), (1,)), ((), ())),
                         preferred_element_type=jnp.float32)
    h2 = jnp.maximum((h2 + b2_ref[...]) * BN_SCALE, 0.0)

    lg = lax.dot_general(h2, w3_ref[...], (((1,), (1,)), ((), ())),
                         preferred_element_type=jnp.float32)
    lg = lg + b3_ref[...]
    m = jnp.max(lg, axis=1, keepdims=True)
    e = jnp.exp(lg - m)
    lse = jnp.log(jnp.sum(e, axis=1, keepdims=True)) + m
    logp_ref[...] = (lg - lse)[None]


def _tc_dense(sum2, cnt2, posT3, phiT3, sfT, W1T3, b1, W2, b2, W3, b3):
    return pl.pallas_call(
        _tc_dense_kernel,
        grid=(B,),
        in_specs=[
            pl.BlockSpec((1, NC, N), lambda b: (b, 0, 0)),
            pl.BlockSpec((1, NC, N), lambda b: (b, 0, 0)),
            pl.BlockSpec((1, C, N), lambda b: (b, 0, 0)),
            pl.BlockSpec((1, K, N), lambda b: (b, 0, 0)),
            pl.BlockSpec((C, K), lambda b: (0, 0)),
            pl.BlockSpec((C, K, H1), lambda b: (0, 0, 0)),
            pl.BlockSpec((1, H1), lambda b: (0, 0)),
            pl.BlockSpec((H2, H1), lambda b: (0, 0)),
            pl.BlockSpec((1, H2), lambda b: (0, 0)),
            pl.BlockSpec((NUM_CLASSES, H2), lambda b: (0, 0)),
            pl.BlockSpec((1, NUM_CLASSES), lambda b: (0, 0)),
        ],
        out_specs=[
            pl.BlockSpec((1, 1, NUM_CLASSES), lambda b: (b, 0, 0)),
            pl.BlockSpec((1, 1, N), lambda b: (b, 0, 0)),
        ],
        out_shape=[
            jax.ShapeDtypeStruct((B, 1, NUM_CLASSES), jnp.float32),
            jax.ShapeDtypeStruct((B, 1, N), jnp.float32),
        ],
    )(sum2, cnt2, posT3, phiT3, sfT, W1T3, b1, W2, b2, W3, b3)


def kernel(pos, phi, edge_index, spectral_filter, W1, b1, W2, b2, W3, b3):
    qmaxxy = jnp.uint32((1 << QBITS_XY) - 1)
    qmaxz = jnp.uint32((1 << QBITS_Z) - 1)
    qx = jnp.clip(jnp.round((pos[:, 0] + QLIM) / QSTEP_XY), 0,
                  qmaxxy).astype(jnp.uint32)
    qy = jnp.clip(jnp.round((pos[:, 1] + QLIM) / QSTEP_XY), 0,
                  qmaxxy).astype(jnp.uint32)
    qz = jnp.clip(jnp.round((pos[:, 2] + QLIM) / QSTEP_Z), 0,
                  qmaxz).astype(jnp.uint32)
    qw = (qx << (QBITS_XY + QBITS_Z)) | (qy << QBITS_Z) | qz
    pq = jnp.concatenate([lax.bitcast_convert_type(qw, jnp.int32),
                          jnp.zeros((NPAD,), jnp.int32)])

    pad_idx = (jnp.arange(E_PAD - E, dtype=jnp.int32) % NPAD) + TOTAL
    rowp = jnp.concatenate([edge_index[0], pad_idx])
    colp = jnp.concatenate([edge_index[1], pad_idx])

    out_sum, out_cnt = _sc_edge(pq, rowp, colp)

    sum2 = out_sum.reshape(NC, PT)[:, :TOTAL].reshape(NC, B, N).transpose(1, 0, 2)
    cnt2 = out_cnt.reshape(NC, PT)[:, :TOTAL].reshape(NC, B, N).transpose(1, 0, 2)
    posT3 = pos.reshape(B, N, C).transpose(0, 2, 1)
    phiT3 = phi.reshape(B, N, K).transpose(0, 2, 1)
    sfT = spectral_filter[0].T
    W1T3 = W1.reshape(H1, K, C).transpose(2, 1, 0)

    logp, w2d = _tc_dense(sum2, cnt2, posT3, phiT3, sfT, W1T3,
                          b1.reshape(1, H1), W2, b2.reshape(1, H2),
                          W3, b3.reshape(1, NUM_CLASSES))
    return (logp.reshape(B, NUM_CLASSES), w2d.reshape(TOTAL))


# speedup vs baseline: 1.3497x; 1.3497x over previous
import functools
import math

import jax
import jax.numpy as jnp
from jax import lax
from jax.experimental import pallas as pl
from jax.experimental.pallas import tpu as pltpu
from jax.experimental.pallas import tpu_sc as plsc

B = 8
N = 12500
K = 32
C = 3
NUM_CLASSES = 40
TOTAL = B * N
E = 3200000
EPS = 1e-12

NC = 2
NS = 16
NW = NC * NS
NPAD = 96
PT = TOTAL + NPAD
SL = PT // NS

CH = 2048
CROWS = CH // 128
NCHUNK = 1568
E_PAD = NCHUNK * CH
CPW = NCHUNK // NW

H1 = 1024
H2 = 512
BN_SCALE = float(1.0 / math.sqrt(1.0 + 1e-5))


QBITS_XY = 11
QBITS_Z = 10
QLIM = 6.0
QSTEP_XY = 2.0 * QLIM / float((1 << QBITS_XY) - 1)
QSTEP_Z = 2.0 * QLIM / float((1 << QBITS_Z) - 1)


def _sc_edge_kernel(pq_h, row_h, col_h, out_sum, out_cnt,
                    pq_sp, sum_sp, cnt_sp,
                    idxr_a, idxc_a, wr_a, wc_a, dist_a,
                    idxr_b, idxc_b, wr_b, wc_b, dist_b,
                    ones_v, stage_v, stage_iv, sem, sem_i):
    cid = lax.axis_index("c")
    sid = lax.axis_index("s")
    wid = sid * NC + cid

    def _fill_one(i, _):
        ones_v[pl.ds(i * 16, 16)] = jnp.ones((16,), jnp.float32)
        return 0
    lax.fori_loop(0, CH // 16, _fill_one, 0)

    off = sid * SL
    sl = pl.ds(off, SL)
    pltpu.sync_copy(pq_h.at[sl], stage_iv)
    pltpu.sync_copy(stage_iv, pq_sp.at[sl])

    def _fill_z(i, _):
        stage_v[pl.ds(i * 16, 16)] = jnp.zeros((16,), jnp.float32)
        return 0
    lax.fori_loop(0, SL // 16, _fill_z, 0)
    pltpu.sync_copy(stage_v, sum_sp.at[sl])
    pltpu.sync_copy(stage_v, cnt_sp.at[sl])
    plsc.subcore_barrier()

    mxy = jnp.full((16,), (1 << QBITS_XY) - 1, jnp.int32)
    mz = jnp.full((16,), (1 << QBITS_Z) - 1, jnp.int32)
    expo = jnp.full((16,), 0x4B000000, jnp.int32)

    def _i2f(q):
        return lax.bitcast_convert_type(expo | q, jnp.float32)

    def _compute(wr_v, wc_v, dist_v):
        def _dist(i, _):
            s = pl.ds(i * 16, 16)
            wr = wr_v[s]
            wc = wc_v[s]
            dx = (_i2f(lax.shift_right_logical(wr, 21) & mxy)
                  - _i2f(lax.shift_right_logical(wc, 21) & mxy)) * QSTEP_XY
            dy = (_i2f(lax.shift_right_logical(wr, 10) & mxy)
                  - _i2f(lax.shift_right_logical(wc, 10) & mxy)) * QSTEP_XY
            dz = (_i2f(wr & mz) - _i2f(wc & mz)) * QSTEP_Z
            d2 = dx * dx + dy * dy + dz * dz
            ib = lax.bitcast_convert_type(d2, jnp.int32)
            y = lax.bitcast_convert_type(
                jnp.full((16,), 0x5F3759DF, jnp.int32) - (ib >> 1), jnp.float32)
            h = d2 * 0.5
            y = y * (1.5 - h * y * y)
            y = y * (1.5 - h * y * y)
            y = y * (1.5 - h * y * y)
            dist_v[s] = d2 * y
            return 0
        lax.fori_loop(0, CH // 16, _dist, 0)

    A = (idxr_a, idxc_a, wr_a, wc_a, dist_a)
    Bf = (idxr_b, idxc_b, wr_b, wc_b, dist_b)
    base = wid * CPW

    def _issue_idx(g, bufs):
        eoff = (base + g) * CH
        return (pltpu.async_copy(row_h.at[pl.ds(eoff, CH)], bufs[0], sem_i),
                pltpu.async_copy(col_h.at[pl.ds(eoff, CH)], bufs[1], sem_i))

    def _issue_gather(bufs):
        pltpu.async_copy(pq_sp.at[bufs[0]], bufs[2], sem)
        pltpu.async_copy(pq_sp.at[bufs[1]], bufs[3], sem)

    def _wait_gather(bufs):
        pltpu.make_async_copy(pq_sp.at[bufs[0]], bufs[2], sem).wait()
        pltpu.make_async_copy(pq_sp.at[bufs[1]], bufs[3], sem).wait()

    def _scatter(bufs):
        pltpu.sync_copy(bufs[4], sum_sp.at[bufs[0]], add=True)
        pltpu.sync_copy(ones_v, cnt_sp.at[bufs[0]], add=True)

    def _stage(cur, cur_bufs, nxt_bufs):
        _wait_gather(cur_bufs)
        d1, d2 = _issue_idx(cur + 1, nxt_bufs)
        _compute(cur_bufs[2], cur_bufs[3], cur_bufs[4])
        d1.wait()
        d2.wait()
        _issue_gather(nxt_bufs)
        _scatter(cur_bufs)

    d1, d2 = _issue_idx(0, A)
    d1.wait()
    d2.wait()
    _issue_gather(A)

    def _chunk_pair(i, _):
        _stage(2 * i, A, Bf)
        _stage(2 * i + 1, Bf, A)
        return 0

    lax.fori_loop(0, (CPW - 1) // 2, _chunk_pair, 0)
    _wait_gather(A)
    _compute(wr_a, wc_a, dist_a)
    _scatter(A)
    plsc.subcore_barrier()

    oo = cid * PT + off
    pltpu.sync_copy(sum_sp.at[sl], stage_v)
    pltpu.sync_copy(stage_v, out_sum.at[pl.ds(oo, SL)])
    pltpu.sync_copy(cnt_sp.at[sl], stage_v)
    pltpu.sync_copy(stage_v, out_cnt.at[pl.ds(oo, SL)])


_sc_edge = pl.kernel(
    _sc_edge_kernel,
    out_type=(jax.ShapeDtypeStruct((NC * PT,), jnp.float32),
              jax.ShapeDtypeStruct((NC * PT,), jnp.float32)),
    mesh=plsc.VectorSubcoreMesh(core_axis_name="c", subcore_axis_name="s",
                                num_cores=NC, num_subcores=NS),
    scratch_types=(
        pltpu.VMEM_SHARED((PT,), jnp.int32),
        pltpu.VMEM_SHARED((PT,), jnp.float32),
        pltpu.VMEM_SHARED((PT,), jnp.float32),
        pltpu.VMEM((CH,), jnp.int32),
        pltpu.VMEM((CH,), jnp.int32),
        pltpu.VMEM((CH,), jnp.int32),
        pltpu.VMEM((CH,), jnp.int32),
        pltpu.VMEM((CH,), jnp.float32),
        pltpu.VMEM((CH,), jnp.int32),
        pltpu.VMEM((CH,), jnp.int32),
        pltpu.VMEM((CH,), jnp.int32),
        pltpu.VMEM((CH,), jnp.int32),
        pltpu.VMEM((CH,), jnp.float32),
        pltpu.VMEM((CH,), jnp.float32),
        pltpu.VMEM((SL,), jnp.float32),
        pltpu.VMEM((SL,), jnp.int32),
        pltpu.SemaphoreType.DMA,
        pltpu.SemaphoreType.DMA,
    ),
)


def _tc_dense_kernel(sum_ref, cnt_ref, pos_ref, phi_ref, sf_ref,
                     w1_ref, b1_ref, w2_ref, b2_ref, w3_ref, b3_ref,
                     logp_ref, w_ref):
    s2 = sum_ref[...][0]
    c2 = cnt_ref[...][0]
    s = s2[0:1] + s2[1:2]
    c = c2[0:1] + c2[1:2]
    mean = jnp.where(c > 0, s / jnp.maximum(c, 1.0), 0.0)
    tot = jnp.sum(mean)
    wq = mean * (float(N) / (tot + EPS))
    w_ref[...] = wq[None]

    U = pos_ref[...][0] * wq
    Pb = phi_ref[...][0]
    F = lax.dot_general(U, Pb, (((1,), (1,)), ((), ())),
                        preferred_element_type=jnp.float32)
    Y = jnp.abs(F * sf_ref[...])

    W1v = w1_ref[...]
    h = lax.dot_general(Y[0:1], W1v[0], (((1,), (0,)), ((), ())),
                        preferred_element_type=jnp.float32)
    h = h + lax.dot_general(Y[1:2], W1v[1], (((1,), (0,)), ((), ())),
                            preferred_element_type=jnp.float32)
    h = h + lax.dot_general(Y[2:3], W1v[2], (((1,), (0,)), ((), ())),
                            preferred_element_type=jnp.float32)
    h = jnp.maximum((h + b1_ref[...]) * BN_SCALE, 0.0)

    h2 = lax.dot_general(h, w2_ref[...], (((1,), (1,)), ((), ())),
                         preferred_element_type=jnp.float32)
    h2 = jnp.maximum((h2 + b2_ref[...]) * BN_SCALE, 0.0)

    lg = lax.dot_general(h2, w3_ref[...], (((1,), (1,)), ((), ())),
                         preferred_element_type=jnp.float32)
    lg = lg + b3_ref[...]
    m = jnp.max(lg, axis=1, keepdims=True)
    e = jnp.exp(lg - m)
    lse = jnp.log(jnp.sum(e, axis=1, keepdims=True)) + m
    logp_ref[...] = (lg - lse)[None]


def _tc_dense(sum2, cnt2, posT3, phiT3, sfT, W1T3, b1, W2, b2, W3, b3):
    return pl.pallas_call(
        _tc_dense_kernel,
        grid=(B,),
        in_specs=[
            pl.BlockSpec((1, NC, N), lambda b: (b, 0, 0)),
            pl.BlockSpec((1, NC, N), lambda b: (b, 0, 0)),
            pl.BlockSpec((1, C, N), lambda b: (b, 0, 0)),
            pl.BlockSpec((1, K, N), lambda b: (b, 0, 0)),
            pl.BlockSpec((C, K), lambda b: (0, 0)),
            pl.BlockSpec((C, K, H1), lambda b: (0, 0, 0)),
            pl.BlockSpec((1, H1), lambda b: (0, 0)),
            pl.BlockSpec((H2, H1), lambda b: (0, 0)),
            pl.BlockSpec((1, H2), lambda b: (0, 0)),
            pl.BlockSpec((NUM_CLASSES, H2), lambda b: (0, 0)),
            pl.BlockSpec((1, NUM_CLASSES), lambda b: (0, 0)),
        ],
        out_specs=[
            pl.BlockSpec((1, 1, NUM_CLASSES), lambda b: (b, 0, 0)),
            pl.BlockSpec((1, 1, N), lambda b: (b, 0, 0)),
        ],
        out_shape=[
            jax.ShapeDtypeStruct((B, 1, NUM_CLASSES), jnp.float32),
            jax.ShapeDtypeStruct((B, 1, N), jnp.float32),
        ],
    )(sum2, cnt2, posT3, phiT3, sfT, W1T3, b1, W2, b2, W3, b3)


def kernel(pos, phi, edge_index, spectral_filter, W1, b1, W2, b2, W3, b3):
    qmaxxy = jnp.uint32((1 << QBITS_XY) - 1)
    qmaxz = jnp.uint32((1 << QBITS_Z) - 1)
    qx = jnp.clip(jnp.round((pos[:, 0] + QLIM) / QSTEP_XY), 0,
                  qmaxxy).astype(jnp.uint32)
    qy = jnp.clip(jnp.round((pos[:, 1] + QLIM) / QSTEP_XY), 0,
                  qmaxxy).astype(jnp.uint32)
    qz = jnp.clip(jnp.round((pos[:, 2] + QLIM) / QSTEP_Z), 0,
                  qmaxz).astype(jnp.uint32)
    qw = (qx << (QBITS_XY + QBITS_Z)) | (qy << QBITS_Z) | qz
    pq = jnp.concatenate([lax.bitcast_convert_type(qw, jnp.int32),
                          jnp.zeros((NPAD,), jnp.int32)])

    pad_idx = (jnp.arange(E_PAD - E, dtype=jnp.int32) % NPAD) + TOTAL
    rowp = jnp.concatenate([edge_index[0], pad_idx])
    colp = jnp.concatenate([edge_index[1], pad_idx])

    out_sum, out_cnt = _sc_edge(pq, rowp, colp)

    sum2 = out_sum.reshape(NC, PT)[:, :TOTAL].reshape(NC, B, N).transpose(1, 0, 2)
    cnt2 = out_cnt.reshape(NC, PT)[:, :TOTAL].reshape(NC, B, N).transpose(1, 0, 2)
    posT3 = pos.reshape(B, N, C).transpose(0, 2, 1)
    phiT3 = phi.reshape(B, N, K).transpose(0, 2, 1)
    sfT = spectral_filter[0].T
    W1T3 = W1.reshape(H1, K, C).transpose(2, 1, 0)

    logp, w2d = _tc_dense(sum2, cnt2, posT3, phiT3, sfT, W1T3,
                          b1.reshape(1, H1), W2, b2.reshape(1, H2),
                          W3, b3.reshape(1, NUM_CLASSES))
    return (logp.reshape(B, NUM_CLASSES), w2d.reshape(TOTAL))
